# baseline (speedup 1.0000x reference)
import jax
import jax.numpy as jnp
from jax import lax
from jax.experimental import pallas as pl
from jax.experimental.pallas import tpu as pltpu
from jax.experimental.pallas import tpu_sc as plsc

NC = 2
NS = 16
NW = NC * NS
LANES = 16

VOCAB = 1000000
EMBED = 64
ROWS = 4096
COLS = 200
CHUNK = 128
NBUF = 8
NT = 2
NGROUPS = COLS // NBUF
PITCH = CHUNK + 1
SCALE = float(EMBED) ** 0.5

RSTARTS = list(range(0, COLS - LANES + 1, LANES)) + [COLS - LANES]


def _body(x_hbm, table_hbm, out_hbm, idx_v, tmp_v, buf_v, bufT_v, gsem, osem):
    wid = lax.axis_index("s") * NC + lax.axis_index("c")
    iota = lax.broadcasted_iota(jnp.int32, (LANES,), 0)

    def scatter_block(t, r0):
        r_ids = jnp.full((LANES,), r0, jnp.int32) + iota

        def a_body(a, carry):
            v = tmp_v[t, a, :]
            plsc.store_scatter(
                idx_v, [r_ids, jnp.full((LANES,), a, jnp.int32)], v
            )
            return carry
        lax.fori_loop(0, CHUNK, a_body, 0, unroll=4)

    def wait_tmp(t):
        pltpu.make_async_copy(
            x_hbm.at[wid, :, pl.ds(0, LANES)], tmp_v.at[t], gsem
        ).wait()

    for i, r0 in enumerate(RSTARTS):
        t = i % 2
        pltpu.make_async_copy(
            x_hbm.at[wid, :, pl.ds(r0, LANES)], tmp_v.at[t], gsem
        ).start()
        if i > 0:
            wait_tmp(1 - t)
            scatter_block(1 - t, RSTARTS[i - 1])
    wait_tmp((len(RSTARTS) - 1) % 2)
    scatter_block((len(RSTARTS) - 1) % 2, RSTARTS[-1])

    def fire_gather(r, rr):
        pltpu.make_async_copy(
            table_hbm.at[idx_v.at[r, pl.ds(0, CHUNK)]], buf_v.at[rr], gsem
        ).start()

    def wait_gather(rr):
        pltpu.make_async_copy(
            table_hbm.at[idx_v.at[0, pl.ds(0, CHUNK)]], buf_v.at[rr], gsem
        ).wait()

    def wait_one_out():
        pltpu.make_async_copy(
            bufT_v.at[0, :, :, pl.ds(0, CHUNK)], out_hbm.at[0, :, wid], osem
        ).wait()

    cblocks = [((c0 + iota) // 8, (c0 + iota) % 8)
               for c0 in range(0, EMBED, LANES)]

    def transpose_scale(rr, tt):
        def a_body(a, carry):
            a_splat = jnp.full((LANES,), a, jnp.int32)
            for cb, (ct_i, cs_i) in enumerate(cblocks):
                v = buf_v[rr, a, pl.ds(cb * LANES, LANES)]
                plsc.store_scatter(
                    bufT_v.at[tt], [ct_i, cs_i, a_splat], v * SCALE
                )
            return carry
        lax.fori_loop(0, CHUNK, a_body, 0, unroll=2)

    def group(g, carry):
        for rr in range(NBUF):
            r = g * NBUF + rr
            tt = rr % NT
            wait_gather(rr)
            pl.when(r >= NT)(wait_one_out)
            transpose_scale(rr, tt)
            pltpu.make_async_copy(
                bufT_v.at[tt, :, :, pl.ds(0, CHUNK)],
                out_hbm.at[r, :, wid],
                osem,
            ).start()
            pl.when(r + NBUF < COLS)(lambda: fire_gather(r + NBUF, rr))
        return carry

    for rr in range(NBUF):
        fire_gather(rr, rr)
    lax.fori_loop(0, NGROUPS, group, 0)
    for _ in range(NT):
        wait_one_out()


def kernel(x, table):
    xs = x.reshape(NW, CHUNK, COLS).astype(jnp.int32)
    o5 = pl.kernel(
        _body,
        out_type=jax.ShapeDtypeStruct((COLS, 8, NW, 8, CHUNK), jnp.float32),
        mesh=plsc.VectorSubcoreMesh(core_axis_name="c", subcore_axis_name="s"),
        scratch_types=[
            pltpu.VMEM((COLS, PITCH), jnp.int32),
            pltpu.VMEM((2, CHUNK, LANES), jnp.int32),
            pltpu.VMEM((NBUF, CHUNK, EMBED), jnp.float32),
            pltpu.VMEM((NT, 8, 8, PITCH), jnp.float32),
            pltpu.SemaphoreType.DMA,
            pltpu.SemaphoreType.DMA,
        ],
        compiler_params=pltpu.CompilerParams(
            use_tc_tiling_on_sc=False, needs_layout_passes=False
        ),
    )(xs, table)
    return o5.transpose(2, 4, 0, 1, 3).reshape(ROWS, COLS, EMBED)


# speedup vs baseline: 1.0708x; 1.0708x over previous
import jax
import jax.numpy as jnp
from jax import lax
from jax.experimental import pallas as pl
from jax.experimental.pallas import tpu as pltpu
from jax.experimental.pallas import tpu_sc as plsc

NC = 2
NS = 16
NW = NC * NS
LANES = 16

VOCAB = 1000000
EMBED = 64
ROWS = 4096
COLS = 200
CHUNK = 128
NBUF = 4
NT = 2
NGROUPS = COLS // NBUF
PITCH = CHUNK + 1
SCALE = float(EMBED) ** 0.5

NBLK = VOCAB // CHUNK
BPW = (NBLK + NW - 1) // NW
VTAIL = NBLK * CHUNK


def _relayout_body(tt_hbm, tail_hbm, t2_hbm, tin_v, tout_v, rsem, wsem):
    wid = lax.axis_index("s") * NC + lax.axis_index("c")
    iota = lax.broadcasted_iota(jnp.int32, (LANES,), 0)

    def blk(k):
        return jnp.minimum(wid + NW * k, NBLK - 1)

    def fire_read(k, t):
        pltpu.make_async_copy(
            tt_hbm.at[:, pl.ds(blk(k) * CHUNK, CHUNK)], tin_v.at[t], rsem
        ).start()

    def wait_read(t):
        pltpu.make_async_copy(
            tt_hbm.at[:, pl.ds(0, CHUNK)], tin_v.at[t], rsem
        ).wait()

    def wait_write(t):
        pltpu.make_async_copy(
            tout_v.at[t], t2_hbm.at[pl.ds(0, EMBED)], wsem
        ).wait()

    def transpose(t):
        def d_body(d, carry):
            perm = (iota + d) & (LANES - 1)
            cvecs = [perm + cb * LANES for cb in range(EMBED // LANES)]

            def j_body(j, carry2):
                row = iota + j * LANES
                jrow = jax.lax.shift_right_logical(row, 1)
                hrow = jax.lax.shift_left(row & 1, 6)
                for c_v in cvecs:
                    v = plsc.load_gather(tin_v.at[t], [c_v, row])
                    plsc.store_scatter(
                        tout_v.at[t], [jrow, hrow + c_v], v
                    )
                return carry2
            lax.fori_loop(0, CHUNK // LANES, j_body, 0, unroll=2)
            return carry
        lax.fori_loop(0, LANES, d_body, 0)

    fire_read(0, 0)

    def step(k, carry):
        t = lax.rem(k, 2)
        tn = 1 - t
        wait_read(t)
        pl.when(k + 1 < BPW)(lambda: fire_read(k + 1, tn))
        pl.when(k >= 2)(lambda: wait_write(t))
        transpose(t)
        pltpu.make_async_copy(
            tout_v.at[t], t2_hbm.at[pl.ds(blk(k) * EMBED, EMBED)], wsem
        ).start()
        return carry

    lax.fori_loop(0, BPW, step, 0)
    wait_write(0)
    wait_write(1)

    @pl.when(wid == 0)
    def _():
        pltpu.sync_copy(tail_hbm, t2_hbm.at[pl.ds(VTAIL // 2, 32)])


def _gather_body(x_hbm, table_hbm, out_hbm, idx_v, buf_v, bufT_v, gsem, osem):
    wid = lax.axis_index("s") * NC + lax.axis_index("c")
    iota = lax.broadcasted_iota(jnp.int32, (LANES,), 0)

    pltpu.sync_copy(x_hbm.at[wid], idx_v)

    def fire_gather(r, rr):
        pltpu.make_async_copy(
            table_hbm.at[idx_v.at[r]], buf_v.at[rr], gsem
        ).start()

    def wait_gather(rr):
        pltpu.make_async_copy(
            table_hbm.at[idx_v.at[0]], buf_v.at[rr], gsem
        ).wait()

    def wait_one_out():
        pltpu.make_async_copy(
            bufT_v.at[0, :, :, pl.ds(0, CHUNK)], out_hbm.at[0, :, wid], osem
        ).wait()

    cblocks = [((c0 + iota) // 8, (c0 + iota) % 8)
               for c0 in range(0, EMBED, LANES)]

    def transpose_scale(rr, tt):
        def a_body(a, carry):
            a_splat = jnp.full((LANES,), a, jnp.int32)
            for cb, (ct_i, cs_i) in enumerate(cblocks):
                v = buf_v[rr, a, pl.ds(cb * LANES, LANES)]
                plsc.store_scatter(
                    bufT_v.at[tt], [ct_i, cs_i, a_splat], v * SCALE
                )
            return carry
        lax.fori_loop(0, CHUNK, a_body, 0, unroll=8)

    def group(g, carry):
        for rr in range(NBUF):
            r = g * NBUF + rr
            tt = rr % NT
            wait_gather(rr)
            pl.when(r >= NT)(wait_one_out)
            transpose_scale(rr, tt)
            pltpu.make_async_copy(
                bufT_v.at[tt, :, :, pl.ds(0, CHUNK)],
                out_hbm.at[r, :, wid],
                osem,
            ).start()
            pl.when(r + NBUF < COLS)(lambda: fire_gather(r + NBUF, rr))
        return carry

    for rr in range(NBUF):
        fire_gather(rr, rr)
    lax.fori_loop(0, NGROUPS, group, 0)
    for _ in range(NT):
        wait_one_out()


def kernel(x, table):
    mesh = plsc.VectorSubcoreMesh(core_axis_name="c", subcore_axis_name="s")

    tt = table.T
    tail = table[VTAIL:, :].reshape(32, 2 * EMBED)
    t2 = pl.kernel(
        _relayout_body,
        out_type=jax.ShapeDtypeStruct((VOCAB // 2, 2 * EMBED), jnp.float32),
        mesh=mesh,
        scratch_types=[
            pltpu.VMEM((2, EMBED, CHUNK), jnp.float32),
            pltpu.VMEM((2, EMBED, CHUNK), jnp.float32),
            pltpu.SemaphoreType.DMA,
            pltpu.SemaphoreType.DMA,
        ],
        compiler_params=pltpu.CompilerParams(
            use_tc_tiling_on_sc=True, needs_layout_passes=False
        ),
    )(tt, tail)

    xst = x.reshape(NW, CHUNK, COLS).transpose(0, 2, 1).astype(jnp.int32)
    o5 = pl.kernel(
        _gather_body,
        out_type=jax.ShapeDtypeStruct((COLS, 8, NW, 8, CHUNK), jnp.float32),
        mesh=mesh,
        scratch_types=[
            pltpu.VMEM((COLS, CHUNK), jnp.int32),
            pltpu.VMEM((NBUF, CHUNK, EMBED), jnp.float32),
            pltpu.VMEM((NT, 8, 8, PITCH), jnp.float32),
            pltpu.SemaphoreType.DMA,
            pltpu.SemaphoreType.DMA,
        ],
        compiler_params=pltpu.CompilerParams(
            use_tc_tiling_on_sc=False, needs_layout_passes=False
        ),
    )(xst, t2.reshape(VOCAB, EMBED))
    return o5.transpose(2, 4, 0, 1, 3).reshape(ROWS, COLS, EMBED)


# speedup vs baseline: 2.4657x; 2.3028x over previous
import jax
import jax.numpy as jnp
from jax import lax
from jax.experimental import pallas as pl
from jax.experimental.pallas import tpu as pltpu
from jax.experimental.pallas import tpu_sc as plsc

NC = 2
NS = 16
NW = NC * NS
LANES = 16

VOCAB = 1000000
EMBED = 64
ROWS = 4096
COLS = 200
CHUNK = 128
NBUF = 4
NT = 2
NGROUPS = COLS // NBUF
PITCH = CHUNK + 1
SCALE = float(EMBED) ** 0.5

NBLK = VOCAB // CHUNK
BPW = (NBLK + NW - 1) // NW
VTAIL = NBLK * CHUNK


def _relayout_body(tt_hbm, tail_hbm, t2_hbm, tin_v, tout_v, rsem, wsem):
    wid = lax.axis_index("s") * NC + lax.axis_index("c")
    iota = lax.broadcasted_iota(jnp.int32, (LANES,), 0)

    def blk(k):
        return jnp.minimum(wid + NW * k, NBLK - 1)

    def fire_read(k, t):
        pltpu.make_async_copy(
            tt_hbm.at[:, pl.ds(blk(k) * CHUNK, CHUNK)], tin_v.at[t], rsem
        ).start()

    def wait_read(t):
        pltpu.make_async_copy(
            tt_hbm.at[:, pl.ds(0, CHUNK)], tin_v.at[t], rsem
        ).wait()

    def wait_write(t):
        pltpu.make_async_copy(
            tout_v.at[t], t2_hbm.at[pl.ds(0, EMBED)], wsem
        ).wait()

    def transpose(t):
        @plsc.parallel_loop(0, LANES)
        def d_body(d):
            perm = (iota + d) & (LANES - 1)
            cvecs = [perm + cb * LANES for cb in range(EMBED // LANES)]

            @plsc.parallel_loop(0, CHUNK // LANES, unroll=2)
            def j_body(j):
                row = iota + j * LANES
                jrow = jax.lax.shift_right_logical(row, 1)
                hrow = jax.lax.shift_left(row & 1, 6)
                for c_v in cvecs:
                    v = plsc.load_gather(tin_v.at[t], [c_v, row])
                    plsc.store_scatter(
                        tout_v.at[t], [jrow, hrow + c_v], v
                    )

    fire_read(0, 0)

    def step(k, carry):
        t = lax.rem(k, 2)
        tn = 1 - t
        wait_read(t)
        pl.when(k + 1 < BPW)(lambda: fire_read(k + 1, tn))
        pl.when(k >= 2)(lambda: wait_write(t))
        transpose(t)
        pltpu.make_async_copy(
            tout_v.at[t], t2_hbm.at[pl.ds(blk(k) * EMBED, EMBED)], wsem
        ).start()
        return carry

    lax.fori_loop(0, BPW, step, 0)
    wait_write(0)
    wait_write(1)

    @pl.when(wid == 0)
    def _():
        pltpu.sync_copy(tail_hbm, t2_hbm.at[pl.ds(VTAIL // 2, 32)])


def _gather_body(x_hbm, table_hbm, out_hbm, idx_v, buf_v, bufT_v, gsem, osem):
    wid = lax.axis_index("s") * NC + lax.axis_index("c")
    iota = lax.broadcasted_iota(jnp.int32, (LANES,), 0)

    pltpu.sync_copy(x_hbm.at[wid], idx_v)

    def fire_gather(r, rr):
        pltpu.make_async_copy(
            table_hbm.at[idx_v.at[r]], buf_v.at[rr], gsem
        ).start()

    def wait_gather(rr):
        pltpu.make_async_copy(
            table_hbm.at[idx_v.at[0]], buf_v.at[rr], gsem
        ).wait()

    def wait_one_out():
        pltpu.make_async_copy(
            bufT_v.at[0, :, :, pl.ds(0, CHUNK)], out_hbm.at[0, :, wid], osem
        ).wait()

    cblocks = [((c0 + iota) // 8, (c0 + iota) % 8)
               for c0 in range(0, EMBED, LANES)]

    def transpose_scale(rr, tt):
        @plsc.parallel_loop(0, CHUNK, unroll=8)
        def a_body(a):
            a_splat = jnp.full((LANES,), a, jnp.int32)
            for cb, (ct_i, cs_i) in enumerate(cblocks):
                v = buf_v[rr, a, pl.ds(cb * LANES, LANES)]
                plsc.store_scatter(
                    bufT_v.at[tt], [ct_i, cs_i, a_splat], v * SCALE
                )

    def group(g, carry):
        for rr in range(NBUF):
            r = g * NBUF + rr
            tt = rr % NT
            wait_gather(rr)
            pl.when(r >= NT)(wait_one_out)
            transpose_scale(rr, tt)
            pltpu.make_async_copy(
                bufT_v.at[tt, :, :, pl.ds(0, CHUNK)],
                out_hbm.at[r, :, wid],
                osem,
            ).start()
            pl.when(r + NBUF < COLS)(lambda: fire_gather(r + NBUF, rr))
        return carry

    for rr in range(NBUF):
        fire_gather(rr, rr)
    lax.fori_loop(0, NGROUPS, group, 0)
    for _ in range(NT):
        wait_one_out()


def kernel(x, table):
    mesh = plsc.VectorSubcoreMesh(core_axis_name="c", subcore_axis_name="s")

    tt = table.T
    tail = table[VTAIL:, :].reshape(32, 2 * EMBED)
    t2 = pl.kernel(
        _relayout_body,
        out_type=jax.ShapeDtypeStruct((VOCAB // 2, 2 * EMBED), jnp.float32),
        mesh=mesh,
        scratch_types=[
            pltpu.VMEM((2, EMBED, CHUNK), jnp.float32),
            pltpu.VMEM((2, EMBED, CHUNK), jnp.float32),
            pltpu.SemaphoreType.DMA,
            pltpu.SemaphoreType.DMA,
        ],
        compiler_params=pltpu.CompilerParams(
            use_tc_tiling_on_sc=True, needs_layout_passes=False
        ),
    )(tt, tail)

    xst = x.reshape(NW, CHUNK, COLS).transpose(0, 2, 1).astype(jnp.int32)
    o5 = pl.kernel(
        _gather_body,
        out_type=jax.ShapeDtypeStruct((COLS, 8, NW, 8, CHUNK), jnp.float32),
        mesh=mesh,
        scratch_types=[
            pltpu.VMEM((COLS, CHUNK), jnp.int32),
            pltpu.VMEM((NBUF, CHUNK, EMBED), jnp.float32),
            pltpu.VMEM((NT, 8, 8, PITCH), jnp.float32),
            pltpu.SemaphoreType.DMA,
            pltpu.SemaphoreType.DMA,
        ],
        compiler_params=pltpu.CompilerParams(
            use_tc_tiling_on_sc=False, needs_layout_passes=False
        ),
    )(xst, t2.reshape(VOCAB, EMBED))
    return o5.transpose(2, 4, 0, 1, 3).reshape(ROWS, COLS, EMBED)
